# baseline (speedup 1.0000x reference)
import jax
import jax.numpy as jnp
from jax.experimental import pallas as pl
from jax.experimental.pallas import tpu as pltpu

NA, NK, NM, ND, NG_ = 3, 8, 32, 32, 12
NAK = NA * NK
BLOCK = 128
MERGED = BLOCK * ND


def _unfold_kernel(qT_ref, QT_ref, gG_ref, om_r_ref, om_c_ref, S_ref, et_ref,
                   e_ref, out_ref, P_scr):
    ak = pl.program_id(1)

    maskT = None
    for c in range(3):
        unf = qT_ref[c : c + 1, :] + gG_ref[:, c : c + 1]
        diff = QT_ref[c : c + 1, :] - unf
        cond = jnp.abs(diff) <= 1e-5 + 1e-5 * jnp.abs(unf)
        maskT = cond if maskT is None else jnp.logical_and(maskT, cond)
    wmask = maskT.astype(jnp.float32)

    e_ak = e_ref[0, 0]
    es = jnp.sum(e_ak * wmask[None, :, :], axis=1)

    W = jnp.broadcast_to(es[None, :, :], (BLOCK, ND, BLOCK)).reshape(MERGED, BLOCK)
    B = W * S_ref[...]
    t_m = et_ref[0, 0]
    dots = jnp.dot(t_m, B, preferred_element_type=jnp.float32)
    sq = dots * dots

    @pl.when(ak == 0)
    def _init():
        P_scr[...] = sq

    @pl.when(ak != 0)
    def _acc():
        P_scr[...] = P_scr[...] + sq

    @pl.when(ak == NAK - 1)
    def _finish():
        eq = (om_r_ref[...] == om_c_ref[...]).astype(jnp.float32)
        out_ref[...] = jnp.dot(
            P_scr[...].T, eq, preferred_element_type=jnp.float32
        ) * (4.0 / 12.0)


@jax.jit
def kernel(q, Q, omega, e_tilde, e, g, G):
    nq = q.shape[0]
    qT = q.T
    QT = Q.T
    gG = g - G[None, :]
    om_r = omega.reshape(NM, 1)
    om_c = omega.reshape(1, NM)
    et2 = e_tilde.reshape(NA, NK, NM, nq * ND)
    S = (jnp.arange(MERGED, dtype=jnp.int32)[:, None] // ND
         == jnp.arange(BLOCK, dtype=jnp.int32)[None, :]).astype(jnp.float32)

    grid = (nq // BLOCK, NAK)
    out = pl.pallas_call(
        _unfold_kernel,
        grid=grid,
        in_specs=[
            pl.BlockSpec((3, BLOCK), lambda b, ak: (0, b)),
            pl.BlockSpec((3, BLOCK), lambda b, ak: (0, b)),
            pl.BlockSpec((NG_, 3), lambda b, ak: (0, 0)),
            pl.BlockSpec((NM, 1), lambda b, ak: (0, 0)),
            pl.BlockSpec((1, NM), lambda b, ak: (0, 0)),
            pl.BlockSpec((MERGED, BLOCK), lambda b, ak: (0, 0)),
            pl.BlockSpec((1, 1, NM, MERGED),
                         lambda b, ak: (ak // NK, ak % NK, 0, b)),
            pl.BlockSpec((1, 1, ND, NG_, BLOCK),
                         lambda b, ak: (ak // NK, ak % NK, 0, 0, b)),
        ],
        out_specs=pl.BlockSpec((BLOCK, NM), lambda b, ak: (b, 0)),
        out_shape=jax.ShapeDtypeStruct((nq, NM), jnp.float32),
        scratch_shapes=[pltpu.VMEM((NM, BLOCK), jnp.float32)],
    )(qT, QT, gG, om_r, om_c, S, et2, e)
    return out


# speedup vs baseline: 1.6327x; 1.6327x over previous
import jax
import jax.numpy as jnp
from jax.experimental import pallas as pl
from jax.experimental.pallas import tpu as pltpu

NA, NK, NM, ND, NG_ = 3, 8, 32, 32, 12
NAK = NA * NK
BLOCK = 128
MERGED = BLOCK * ND


def _unfold_kernel(qT_ref, QT_ref, gG_ref, om_r_ref, om_c_ref, S_ref, R_ref,
                   et_ref, e_ref, out_ref, P_scr):
    a = pl.program_id(1)

    maskT = None
    for c in range(3):
        unf = qT_ref[c : c + 1, :] + gG_ref[:, c : c + 1]
        diff = QT_ref[c : c + 1, :] - unf
        cond = jnp.abs(diff) <= 1e-5 + 1e-5 * jnp.abs(unf)
        maskT = cond if maskT is None else jnp.logical_and(maskT, cond)
    wmask = maskT.astype(jnp.float32)

    @pl.when(a == 0)
    def _init():
        P_scr[...] = jnp.zeros_like(P_scr)

    for k in range(NK):
        e_k = e_ref[0, k]
        es = jnp.sum(e_k * wmask[None, :, :], axis=1)
        W = jnp.dot(R_ref[...], es, preferred_element_type=jnp.float32)
        B = W * S_ref[...]
        t_k = et_ref[0, k]
        dots = jnp.dot(t_k, B, preferred_element_type=jnp.float32)
        P_scr[...] = P_scr[...] + dots * dots

    @pl.when(a == NA - 1)
    def _finish():
        eq = (om_r_ref[...] == om_c_ref[...]).astype(jnp.float32)
        out_ref[...] = jnp.dot(
            P_scr[...].T, eq, preferred_element_type=jnp.float32
        ) * (4.0 / 12.0)


@jax.jit
def kernel(q, Q, omega, e_tilde, e, g, G):
    nq = q.shape[0]
    qT = q.T
    QT = Q.T
    gG = g - G[None, :]
    om_r = omega.reshape(NM, 1)
    om_c = omega.reshape(1, NM)
    et2 = e_tilde.reshape(NA, NK, NM, nq * ND)
    rows = jnp.arange(MERGED, dtype=jnp.int32)
    S = (rows[:, None] // ND
         == jnp.arange(BLOCK, dtype=jnp.int32)[None, :]).astype(jnp.float32)
    R = (rows[:, None] % ND
         == jnp.arange(ND, dtype=jnp.int32)[None, :]).astype(jnp.float32)

    grid = (nq // BLOCK, NA)
    out = pl.pallas_call(
        _unfold_kernel,
        grid=grid,
        in_specs=[
            pl.BlockSpec((3, BLOCK), lambda b, a: (0, b)),
            pl.BlockSpec((3, BLOCK), lambda b, a: (0, b)),
            pl.BlockSpec((NG_, 3), lambda b, a: (0, 0)),
            pl.BlockSpec((NM, 1), lambda b, a: (0, 0)),
            pl.BlockSpec((1, NM), lambda b, a: (0, 0)),
            pl.BlockSpec((MERGED, BLOCK), lambda b, a: (0, 0)),
            pl.BlockSpec((MERGED, ND), lambda b, a: (0, 0)),
            pl.BlockSpec((1, NK, NM, MERGED), lambda b, a: (a, 0, 0, b)),
            pl.BlockSpec((1, NK, ND, NG_, BLOCK), lambda b, a: (a, 0, 0, 0, b)),
        ],
        out_specs=pl.BlockSpec((BLOCK, NM), lambda b, a: (b, 0)),
        out_shape=jax.ShapeDtypeStruct((nq, NM), jnp.float32),
        scratch_shapes=[pltpu.VMEM((NM, BLOCK), jnp.float32)],
    )(qT, QT, gG, om_r, om_c, S, R, et2, e)
    return out
